# baseline (speedup 1.0000x reference)
import functools

import jax
import jax.numpy as jnp
from jax import lax
from jax.experimental import pallas as pl
from jax.experimental.pallas import tpu as pltpu
from jax.experimental.pallas import tpu_sc as plsc

F32 = jnp.float32
BF16 = jnp.bfloat16
I32 = jnp.int32

NN = 10000
NE = 160000
B = 256
NP = 12288
EP = 163840
G = 3
GACC = 512
ECH = EP // 128
NCH = NP // 128
BLK = 2048


def _rows(blk, w):
    return pl.BlockSpec((blk, w), lambda i: (i, 0))


def _full(shape):
    return pl.BlockSpec(shape, lambda i: tuple(0 for _ in shape))


def _mesh():
    return plsc.VectorSubcoreMesh(core_axis_name="c", subcore_axis_name="s")


def _sc_gather(a, b, rowp, colp):
    tch = rowp.shape[0]
    pt = tch // 32
    rows = tch * 128

    @functools.partial(
        pl.kernel,
        out_type=jax.ShapeDtypeStruct((rows, 64), BF16),
        mesh=_mesh(),
        compiler_params=pltpu.CompilerParams(use_tc_tiling_on_sc=False),
        scratch_types=[
            pltpu.VMEM((pt, 128), I32),
            pltpu.VMEM((pt, 128), I32),
        ] + [pltpu.VMEM((128, 64), BF16)] * 8
          + [pltpu.SemaphoreType.DMA] * 8,
    )
    def k(a_h, b_h, row_h, col_h, g_h, ridx, cidx,
          a0, a1, a2, a3, b0, b1, b2, b3,
          sg0, sg1, sg2, sg3, sw0, sw1, sw2, sw3):
        ba = (a0, a1, a2, a3)
        bb = (b0, b1, b2, b3)
        sg = (sg0, sg1, sg2, sg3)
        sw = (sw0, sw1, sw2, sw3)
        c = lax.axis_index("c")
        s = lax.axis_index("s")
        wid = c * 16 + s
        pltpu.sync_copy(row_h.at[pl.ds(wid * pt, pt)], ridx)
        pltpu.sync_copy(col_h.at[pl.ds(wid * pt, pt)], cidx)

        def fire_g(j, t):
            pltpu.async_copy(a_h.at[ridx.at[j]], ba[t], sg[t])
            pltpu.async_copy(b_h.at[cidx.at[j]], bb[t], sg[t])

        def wait_g(j, t):
            pltpu.make_async_copy(a_h.at[ridx.at[j]], ba[t], sg[t]).wait()
            pltpu.make_async_copy(b_h.at[cidx.at[j]], bb[t], sg[t]).wait()

        def add_bufs(t):
            def rbody(i, carry):
                for q in range(2):
                    sl_ = pl.ds(q * 32, 32)
                    ba[t][i, sl_] = ba[t][i, sl_] + bb[t][i, sl_]
                return carry

            lax.fori_loop(0, 128, rbody, 0)

        def fire_w(j, t):
            base = pl.multiple_of((wid * pt + j) * 128, 128)
            pltpu.async_copy(ba[t], g_h.at[pl.ds(base, 128)], sw[t])

        def wait_w(j, t):
            base = pl.multiple_of((wid * pt + j) * 128, 128)
            pltpu.make_async_copy(ba[t], g_h.at[pl.ds(base, 128)],
                                  sw[t]).wait()

        fire_g(0, 0)
        fire_g(1, 1)
        fire_g(2, 2)
        wait_g(0, 0)
        add_bufs(0)
        fire_w(0, 0)
        fire_g(3, 3)
        wait_g(1, 1)
        add_bufs(1)
        fire_w(1, 1)

        def body(kk, carry):
            j0 = kk * 4
            for t in range(4):
                j = j0 + t
                wait_w(j - 4, t)
                fire_g(j, t)
                wait_g(j - 2, (t + 2) % 4)
                add_bufs((t + 2) % 4)
                fire_w(j - 2, (t + 2) % 4)
            return carry

        lax.fori_loop(1, pt // 4, body, 0)
        wait_g(pt - 2, (pt - 2) % 4)
        add_bufs((pt - 2) % 4)
        fire_w(pt - 2, (pt - 2) % 4)
        wait_g(pt - 1, (pt - 1) % 4)
        add_bufs((pt - 1) % 4)
        fire_w(pt - 1, (pt - 1) % 4)
        for j in (pt - 4, pt - 3, pt - 2, pt - 1):
            wait_w(j, j % 4)

    return k(a, b, rowp, colp)


def _sc_scatter_edges(ee, colp, zeros_e, btcp=None, ones=None,
                      zc_e=None, zc_n3=None):
    pt = ECH // 32
    rpt = NP // 16
    counts = btcp is not None
    npt = (G * NCH) // 32 if counts else 0
    nrpt = (G * GACC) // 16

    outs = [jax.ShapeDtypeStruct((2, G * NP, 64), F32)]
    scr = ([pltpu.VMEM((G * pt, 128), I32)]
           + [pltpu.VMEM((128, 64), F32)] * 4
           + [pltpu.SemaphoreType.DMA] * 8)
    if counts:
        outs += [jax.ShapeDtypeStruct((2, G * NP, 16), F32),
                 jax.ShapeDtypeStruct((2, G * GACC, 16), F32)]
        scr += [pltpu.VMEM((npt, 128), I32),
                pltpu.VMEM((128, 16), F32),
                pltpu.VMEM_SHARED((NP, 16), F32),
                pltpu.VMEM_SHARED((G * GACC, 16), F32),
                pltpu.SemaphoreType.DMA]
    scr += [pltpu.VMEM_SHARED((NP, 64), F32)]

    @functools.partial(
        pl.kernel,
        out_type=tuple(outs) if counts else outs[0],
        mesh=_mesh(),
        compiler_params=pltpu.CompilerParams(use_tc_tiling_on_sc=False),
        scratch_types=scr,
    )
    def k(*refs):
        if counts:
            (d_h, i_h, z_h, b_h, o_h, ze_h, zn_h, out_h, ec_h, nc_h,
             iidx, d0, d1, d2, d3, sl0, sl1, sl2, sl3, ss0, ss1, ss2, ss3,
             nidx, obuf, eacc, nacc, scnt, acc) = refs
        else:
            (d_h, i_h, z_h, out_h,
             iidx, d0, d1, d2, d3, sl0, sl1, sl2, sl3, ss0, ss1, ss2, ss3,
             acc) = refs
        dbuf = (d0, d1, d2, d3)
        sl = (sl0, sl1, sl2, sl3)
        ss = (ss0, ss1, ss2, ss3)
        c = lax.axis_index("c")
        s = lax.axis_index("s")
        for g in range(G):
            pltpu.sync_copy(
                i_h.at[pl.ds(g * ECH + c * (ECH // 2) + s * pt, pt)],
                iidx.at[pl.ds(g * pt, pt)])
        if counts:
            nstart = c * ((G * NCH) // 2) + s * npt
            pltpu.sync_copy(b_h.at[pl.ds(nstart, npt)], nidx)
            pltpu.sync_copy(o_h, obuf)
            pltpu.sync_copy(zn_h.at[pl.ds(s * nrpt, nrpt)],
                            nacc.at[pl.ds(s * nrpt, nrpt)])

        for g in range(G):
            gbase = g * ECH + c * (ECH // 2) + s * pt

            def fire_l(j, t):
                base = pl.multiple_of((gbase + j) * 128, 128)
                pltpu.async_copy(d_h.at[pl.ds(base, 128)], dbuf[t], sl[t])

            def wait_l(j, t):
                base = pl.multiple_of((gbase + j) * 128, 128)
                pltpu.make_async_copy(d_h.at[pl.ds(base, 128)], dbuf[t],
                                      sl[t]).wait()

            def fire_s(j, t):
                pltpu.async_copy(dbuf[t], acc.at[iidx.at[g * pt + j]],
                                 ss[t], add=True)
                if counts:
                    pltpu.async_copy(obuf, eacc.at[iidx.at[g * pt + j]],
                                     scnt, add=True)

            def wait_s(j, t):
                pltpu.make_async_copy(dbuf[t], acc.at[iidx.at[g * pt + j]],
                                      ss[t]).wait()

            fire_l(0, 0)
            fire_l(1, 1)
            pltpu.sync_copy(z_h.at[pl.ds(s * rpt, rpt)],
                            acc.at[pl.ds(s * rpt, rpt)])
            if counts:
                pltpu.sync_copy(ze_h.at[pl.ds(s * rpt, rpt)],
                                eacc.at[pl.ds(s * rpt, rpt)])
            fire_l(2, 2)
            plsc.subcore_barrier()
            if counts and g == 0:
                for j in range(npt):
                    pltpu.async_copy(obuf, nacc.at[nidx.at[j]], scnt,
                                     add=True)
            wait_l(0, 0)
            fire_s(0, 0)
            fire_l(3, 3)
            wait_l(1, 1)
            fire_s(1, 1)

            def body(kk, carry):
                j0 = kk * 4
                for t in range(4):
                    j = j0 + t
                    wait_s(j - 4, t)
                    fire_l(j, t)
                    wait_l(j - 2, (t + 2) % 4)
                    fire_s(j - 2, (t + 2) % 4)
                return carry

            lax.fori_loop(1, pt // 4, body, 0)
            wait_l(pt - 2, (pt - 2) % 4)
            fire_s(pt - 2, (pt - 2) % 4)
            wait_l(pt - 1, (pt - 1) % 4)
            fire_s(pt - 1, (pt - 1) % 4)
            for j in (pt - 4, pt - 3, pt - 2, pt - 1):
                wait_s(j, j % 4)
            if counts:
                def drain(j, carry):
                    pltpu.make_async_copy(
                        obuf, eacc.at[iidx.at[g * pt + j]], scnt).wait()
                    return carry

                lax.fori_loop(g * pt, (g + 1) * pt, drain, 0)
                if g == 0:
                    for j in range(npt):
                        pltpu.make_async_copy(obuf, nacc.at[nidx.at[j]],
                                              scnt).wait()
            plsc.subcore_barrier()
            pltpu.sync_copy(acc.at[pl.ds(s * rpt, rpt)],
                            out_h.at[c, pl.ds(g * NP + s * rpt, rpt)])
            if counts:
                pltpu.sync_copy(eacc.at[pl.ds(s * rpt, rpt)],
                                ec_h.at[c, pl.ds(g * NP + s * rpt, rpt)])
                if g == 0:
                    pltpu.sync_copy(nacc.at[pl.ds(s * nrpt, nrpt)],
                                    nc_h.at[c, pl.ds(s * nrpt, nrpt)])
            if g + 1 < G:
                plsc.subcore_barrier()

    if counts:
        return k(ee, colp, zeros_e, btcp, ones, zc_e, zc_n3)
    return k(ee, colp, zeros_e)


def _sc_scatter_nodes(x2, btcp, zeros_n3):
    tch = btcp.shape[0]
    pt = tch // 32
    nrpt = (G * GACC) // 16

    @functools.partial(
        pl.kernel,
        out_type=jax.ShapeDtypeStruct((2, G * GACC, 128), F32),
        mesh=_mesh(),
        compiler_params=pltpu.CompilerParams(use_tc_tiling_on_sc=False),
        scratch_types=[pltpu.VMEM((pt, 128), I32)]
        + [pltpu.VMEM((128, 128), F32)] * 4
        + [pltpu.SemaphoreType.DMA] * 8
        + [pltpu.VMEM_SHARED((G * GACC, 128), F32)],
    )
    def k(d_h, i_h, z_h, out_h, iidx, d0, d1, d2, d3,
          sl0, sl1, sl2, sl3, ss0, ss1, ss2, ss3, acc):
        dbuf = (d0, d1, d2, d3)
        sl = (sl0, sl1, sl2, sl3)
        ss = (ss0, ss1, ss2, ss3)
        c = lax.axis_index("c")
        s = lax.axis_index("s")
        start = c * (tch // 2) + s * pt
        pltpu.sync_copy(i_h.at[pl.ds(start, pt)], iidx)

        def fire_l(j, t):
            base = pl.multiple_of((start + j) * 128, 128)
            pltpu.async_copy(d_h.at[pl.ds(base, 128)], dbuf[t], sl[t])

        def wait_l(j, t):
            base = pl.multiple_of((start + j) * 128, 128)
            pltpu.make_async_copy(d_h.at[pl.ds(base, 128)], dbuf[t],
                                  sl[t]).wait()

        def fire_s(j, t):
            pltpu.async_copy(dbuf[t], acc.at[iidx.at[j]], ss[t], add=True)

        def wait_s(j, t):
            pltpu.make_async_copy(dbuf[t], acc.at[iidx.at[j]], ss[t]).wait()

        fire_l(0, 0)
        fire_l(1, 1)
        fire_l(2, 2)
        pltpu.sync_copy(z_h.at[pl.ds(s * nrpt, nrpt)],
                        acc.at[pl.ds(s * nrpt, nrpt)])
        plsc.subcore_barrier()
        wait_l(0, 0)
        fire_s(0, 0)
        fire_l(3, 3)
        wait_l(1, 1)
        fire_s(1, 1)
        for j in range(4, pt):
            t = j % 4
            wait_s(j - 4, t)
            fire_l(j, t)
            wait_l(j - 2, (j - 2) % 4)
            fire_s(j - 2, (j - 2) % 4)
        wait_l(pt - 2, (pt - 2) % 4)
        fire_s(pt - 2, (pt - 2) % 4)
        wait_l(pt - 1, (pt - 1) % 4)
        fire_s(pt - 1, (pt - 1) % 4)
        for j in range(pt - 4, pt):
            wait_s(j, j % 4)
        plsc.subcore_barrier()
        pltpu.sync_copy(acc.at[pl.ds(s * nrpt, nrpt)],
                        out_h.at[c, pl.ds(s * nrpt, nrpt)])

    return k(x2, btcp, zeros_n3)


def _dot(x, w):
    return jnp.dot(x, w, preferred_element_type=F32)


def _tc_enc(x, w1, b1, w2, b2, wa, wb):
    n = x.shape[0]

    def body(x_r, w1_r, b1_r, w2_r, b2_r, wa_r, wb_r, ex_r, a_r, b_r):
        h = jnp.maximum(_dot(x_r[...], w1_r[...]) + b1_r[...], 0.0)
        e = _dot(h, w2_r[...]) + b2_r[...]
        ex_r[...] = e
        a_r[...] = _dot(e, wa_r[...]).astype(BF16)
        b_r[...] = _dot(e, wb_r[...]).astype(BF16)

    return pl.pallas_call(
        body, grid=(n // BLK,),
        in_specs=[_rows(BLK, 128), _full((128, 128)), _full((1, 128)),
                  _full((128, 128)), _full((1, 128)),
                  _full((128, 64)), _full((128, 64))],
        out_specs=[_rows(BLK, 128), _rows(BLK, 64), _rows(BLK, 64)],
        out_shape=[jax.ShapeDtypeStruct((n, 128), F32),
                   jax.ShapeDtypeStruct((n, 64), BF16),
                   jax.ShapeDtypeStruct((n, 64), BF16)],
    )(x, w1, b1, w2, b2, wa, wb)


def _tc_ee(ea8, w1p, b1, w2, b2):
    n = ea8.shape[0]

    def body(e_r, w1_r, b1_r, w2_r, b2_r, o_r):
        h = jnp.maximum(_dot(e_r[...], w1_r[...]) + b1_r[...], 0.0)
        o_r[...] = _dot(h, w2_r[...]) + b2_r[...]

    return pl.pallas_call(
        body, grid=(n // BLK,),
        in_specs=[_rows(BLK, 8), _full((8, 64)), _full((1, 64)),
                  _full((64, 64)), _full((1, 64))],
        out_specs=_rows(BLK, 64),
        out_shape=jax.ShapeDtypeStruct((n, 64), F32),
    )(ea8, w1p, b1, w2, b2)


def _tc_edge(gsum, ee, wc, b1, w2, b2):
    n = gsum.shape[0]

    def body(g_r, e_r, wc_r, b1_r, w2_r, b2_r, o_r):
        h = g_r[...].astype(F32) + _dot(e_r[...], wc_r[...]) + b1_r[...]
        h = jnp.maximum(h, 0.0)
        o_r[...] = _dot(h, w2_r[...]) + b2_r[...]

    return pl.pallas_call(
        body, grid=(n // BLK,),
        in_specs=[_rows(BLK, 64), _rows(BLK, 64),
                  _full((64, 64)), _full((1, 64)),
                  _full((64, 64)), _full((1, 64))],
        out_specs=_rows(BLK, 64),
        out_shape=jax.ShapeDtypeStruct((n, 64), F32),
    )(gsum, ee, wc, b1, w2, b2)


def _tc_node(ex, esum, ecnt, wx, wagg, b1, w2, b2, wa, wb):
    n = ex.shape[0]

    def body(x_r, s0_r, s1_r, c0_r, c1_r, wx_r, wg_r, b1_r, w2_r, b2_r,
             wa_r, wb_r, x2_r, a_r, b_r):
        cnt = c0_r[0][:, 0:1] + c1_r[0][:, 0:1]
        rcp = 1.0 / jnp.maximum(cnt, 1.0)
        agg = (s0_r[0] + s1_r[0]) * rcp
        h = jnp.maximum(_dot(x_r[...], wx_r[...]) + _dot(agg, wg_r[...])
                        + b1_r[...], 0.0)
        x2 = _dot(h, w2_r[...]) + b2_r[...]
        x2_r[...] = x2
        a_r[...] = _dot(x2, wa_r[...]).astype(BF16)
        b_r[...] = _dot(x2, wb_r[...]).astype(BF16)

    return pl.pallas_call(
        body, grid=(n // BLK,),
        in_specs=[_rows(BLK, 128),
                  pl.BlockSpec((1, BLK, 64), lambda i: (0, i, 0)),
                  pl.BlockSpec((1, BLK, 64), lambda i: (1, i, 0)),
                  pl.BlockSpec((1, BLK, 16), lambda i: (0, i, 0)),
                  pl.BlockSpec((1, BLK, 16), lambda i: (1, i, 0)),
                  _full((128, 128)), _full((64, 128)), _full((1, 128)),
                  _full((128, 128)), _full((1, 128)),
                  _full((128, 64)), _full((128, 64))],
        out_specs=[_rows(BLK, 128), _rows(BLK, 64), _rows(BLK, 64)],
        out_shape=[jax.ShapeDtypeStruct((n, 128), F32),
                   jax.ShapeDtypeStruct((n, 64), BF16),
                   jax.ShapeDtypeStruct((n, 64), BF16)],
    )(ex, esum, esum, ecnt, ecnt, wx, wagg, b1, w2, b2, wa, wb)


def _tc_glob(u, nsum, ncnt, wu, wn, b1, w2, b2):
    def body(u_r, s0_r, s1_r, c0_r, c1_r, wu_r, wn_r, b1_r, w2_r, b2_r,
             o_r):
        cnt = c0_r[0][:, 0:1] + c1_r[0][:, 0:1]
        rcp = 1.0 / jnp.maximum(cnt, 1.0)
        nmean = (s0_r[0] + s1_r[0]) * rcp
        h = jnp.maximum(_dot(u_r[...], wu_r[...]) + _dot(nmean, wn_r[...])
                        + b1_r[...], 0.0)
        o_r[...] = _dot(h, w2_r[...]) + b2_r[...]

    return pl.pallas_call(
        body, grid=(G,),
        in_specs=[_rows(B, 64),
                  pl.BlockSpec((1, B, 128), lambda i: (0, 2 * i, 0)),
                  pl.BlockSpec((1, B, 128), lambda i: (1, 2 * i, 0)),
                  pl.BlockSpec((1, B, 16), lambda i: (0, 2 * i, 0)),
                  pl.BlockSpec((1, B, 16), lambda i: (1, 2 * i, 0)),
                  _full((64, 64)), _full((128, 64)), _full((1, 64)),
                  _full((64, 64)), _full((1, 64))],
        out_specs=_rows(B, 64),
        out_shape=jax.ShapeDtypeStruct((G * B, 64), F32),
    )(u, nsum, nsum, ncnt, ncnt, wu, wn, b1, w2, b2)


def _tc_phase2(u1, u2, um, scal4,
               wn21, bn21, wn22, bn22,
               we21, be21, we22, be22,
               wE1, bE1, wE2, bE2,
               wX1, bX1, wX2, bX2,
               wG1, bG1, wG2, bG2,
               wl1, bl1, wl2, bl2):
    def body(u1_r, u2_r, um_r, sc_r,
             wn21_r, bn21_r, wn22_r, bn22_r,
             we21_r, be21_r, we22_r, be22_r,
             wE1_r, bE1_r, wE2_r, bE2_r,
             wX1_r, bX1_r, wX2_r, bX2_r,
             wG1_r, bG1_r, wG2_r, bG2_r,
             wl1_r, bl1_r, wl2_r, bl2_r, o_r):
        sc = sc_r[...]
        t = sc[:, 0:1]
        r1 = sc[:, 1:2] / sc[:, 3:4]
        r2 = sc[:, 2:3] / sc[:, 3:4]
        we21 = we21_r[...]
        h1 = jnp.maximum(t * we21[0:1, :] + r1 * we21[1:2, :] + be21_r[...],
                         0.0)
        c1 = _dot(h1, we22_r[...]) + be22_r[...]
        h2 = jnp.maximum(t * we21[0:1, :] + r2 * we21[1:2, :] + be21_r[...],
                         0.0)
        c2 = _dot(h2, we22_r[...]) + be22_r[...]

        def enc(u):
            h = jnp.maximum(_dot(u, wn21_r[...]) + bn21_r[...], 0.0)
            return _dot(h, wn22_r[...]) + bn22_r[...]

        n0 = enc(u1_r[...])
        n1 = enc(u2_r[...])
        n2 = enc(um_r[...])
        a0 = a1 = c1
        a2 = a3 = c2
        ug = jnp.full((B, 32), 0.1, F32)
        wE1v = wE1_r[...]
        ws, wd, wf = wE1v[0:64], wE1v[64:128], wE1v[128:192]
        wX1v = wX1_r[...]
        wxn, wan = wX1v[0:64], wX1v[64:128]
        wG1v = wG1_r[...]
        wug, wng = wG1v[0:32], wG1v[32:96]
        for _ in range(2):
            def edg(sv, dv, av):
                h = jnp.maximum(_dot(sv, ws) + _dot(dv, wd) + _dot(av, wf)
                                + bE1_r[...], 0.0)
                return _dot(h, wE2_r[...]) + bE2_r[...]

            e0 = edg(n0, n2, a0)
            e1 = edg(n2, n0, a1)
            e2 = edg(n1, n2, a2)
            e3 = edg(n2, n1, a3)

            def nod(v, ag):
                h = jnp.maximum(_dot(v, wxn) + _dot(ag, wan) + bX1_r[...],
                                0.0)
                return _dot(h, wX2_r[...]) + bX2_r[...]

            n0 = nod(n0, e1)
            n1 = nod(n1, e3)
            n2 = nod(n2, (e0 + e2) * 0.5)
            nm = (n0 + n1 + n2) / 3.0
            hg = jnp.maximum(_dot(ug, wug) + _dot(nm, wng) + bG1_r[...], 0.0)
            ug = _dot(hg, wG2_r[...]) + bG2_r[...]
            a0, a1, a2, a3 = e0, e1, e2, e3
        hl = jnp.maximum(_dot(ug, wl1_r[...]) + bl1_r[...], 0.0)
        o_r[...] = _dot(hl, wl2_r[...]) + bl2_r[...]

    return pl.pallas_call(
        body,
        out_shape=jax.ShapeDtypeStruct((B, 15), F32),
    )(u1, u2, um, scal4,
      wn21, bn21, wn22, bn22,
      we21, be21, we22, be22,
      wE1, bE1, wE2, bE2,
      wX1, bX1, wX2, bX2,
      wG1, bG1, wG2, bG2,
      wl1, bl1, wl2, bl2)


def _row2(v):
    return v.reshape(1, -1)


def kernel(x_p1, ei_p1, ea_p1, y_p1, btc_p1, x_p2, ei_p2, ea_p2, y_p2,
           btc_p2, x_pm, ei_pm, ea_pm, y_pm, btc_pm, Temperature, params):
    p = params
    (we1, be1), (we2, be2) = p['enc_node_1']
    (wee1, bee1), (wee2, bee2) = p['enc_edge_1']
    (wE1, bE1), (wE2, bE2) = p['edge1']
    (wN1, bN1), (wN2, bN2) = p['node1']
    (wG1, bG1), (wG2, bG2) = p['glob1']
    wa, wb, wc = wE1[0:128], wE1[128:256], wE1[256:320]
    wx, wagg = wN1[0:128], wN1[128:192]
    wu, wn = wG1[0:64], wG1[64:192]
    wee1p = jnp.concatenate([wee1, jnp.zeros((4, 64), F32)], axis=0)

    zeros_e = jnp.zeros((NP, 64), F32)
    zeros_n3 = jnp.zeros((G * GACC, 128), F32)
    zc_e = jnp.zeros((NP, 16), F32)
    zc_n3 = jnp.zeros((G * GACC, 16), F32)
    ones16 = jnp.ones((128, 16), F32)

    graphs = ((x_p1, ei_p1, ea_p1, btc_p1),
              (x_p2, ei_p2, ea_p2, btc_p2),
              (x_pm, ei_pm, ea_pm, btc_pm))
    rowo, colo, colu, btco, xps, ea8s = [], [], [], [], [], []
    for g, (x, ei, ea, btc) in enumerate(graphs):
        pad_e = jnp.full((EP - NE,), NN, I32)
        rowo.append(jnp.concatenate([ei[0], pad_e]) + g * NP)
        colo.append(jnp.concatenate([ei[1], pad_e]) + g * NP)
        colu.append(jnp.concatenate([ei[1], pad_e]))
        btco.append(jnp.concatenate(
            [btc, jnp.full((NP - NN,), B, I32)]) + g * GACC)
        xps.append(jnp.pad(x, ((0, NP - NN), (0, 0))))
        ea8s.append(jnp.pad(ea, ((0, EP - NE), (0, 4))))
    rowp = jnp.concatenate(rowo).reshape(G * ECH, 128)
    colp = jnp.concatenate(colo).reshape(G * ECH, 128)
    colpu = jnp.concatenate(colu).reshape(G * ECH, 128)
    btcp = jnp.concatenate(btco).reshape(G * NCH, 128)
    xall = jnp.concatenate(xps)
    ea8all = jnp.concatenate(ea8s)

    ex, a, b = _tc_enc(xall, we1, _row2(be1), we2, _row2(be2), wa, wb)
    ee = _tc_ee(ea8all, wee1p, _row2(bee1), wee2, _row2(bee2))
    u = jnp.full((G * B, 64), 0.1, F32)
    ecnt = ncnt = None
    for r in range(2):
        gsum = _sc_gather(a, b, rowp, colp)
        ee = _tc_edge(gsum, ee, wc, _row2(bE1), wE2, _row2(bE2))
        if r == 0:
            esum, ecnt, ncnt = _sc_scatter_edges(
                ee, colpu, zeros_e, btcp, ones16, zc_e, zc_n3)
        else:
            esum = _sc_scatter_edges(ee, colpu, zeros_e)
        ex, a, b = _tc_node(ex, esum, ecnt, wx, wagg, _row2(bN1), wN2,
                            _row2(bN2), wa, wb)
        nsum = _sc_scatter_nodes(ex, btcp, zeros_n3)
        u = _tc_glob(u, nsum, ncnt, wu, wn, _row2(bG1), wG2, _row2(bG2))

    (wn21, bn21), (wn22, bn22) = p['enc_node_2']
    (we21, be21), (we22, be22) = p['enc_edge_2']
    (wE21, bE21), (wE22, bE22) = p['edge2']
    (wX21, bX21), (wX22, bX22) = p['node2']
    (wG21, bG21), (wG22, bG22) = p['glob2']
    (wl1, bl1), (wl2, bl2) = p['last']
    scal4 = jnp.stack([Temperature, y_p1, y_p2, y_pm], axis=1)
    return _tc_phase2(u[0:B], u[B:2 * B], u[2 * B:3 * B], scal4,
                      wn21, _row2(bn21), wn22, _row2(bn22),
                      we21, _row2(be21), we22, _row2(be22),
                      wE21, _row2(bE21), wE22, _row2(bE22),
                      wX21, _row2(bX21), wX22, _row2(bX22),
                      wG21, _row2(bG21), wG22, _row2(bG22),
                      wl1, _row2(bl1), wl2, _row2(bl2))


# speedup vs baseline: 1.0965x; 1.0965x over previous
import functools

import jax
import jax.numpy as jnp
from jax import lax
from jax.experimental import pallas as pl
from jax.experimental.pallas import tpu as pltpu
from jax.experimental.pallas import tpu_sc as plsc

F32 = jnp.float32
BF16 = jnp.bfloat16
I32 = jnp.int32

NN = 10000
NE = 160000
B = 256
NP = 12288
EP = 163840
G = 3
GACC = 512
ECH = EP // 128
NCH = NP // 128
BLK = 2048


def _rows(blk, w):
    return pl.BlockSpec((blk, w), lambda i: (i, 0))


def _full(shape):
    return pl.BlockSpec(shape, lambda i: tuple(0 for _ in shape))


def _mesh():
    return plsc.VectorSubcoreMesh(core_axis_name="c", subcore_axis_name="s")


def _sc_gather(a, b, rowp, colp):
    tch = rowp.shape[0]
    pt = tch // 32
    rows = tch * 128

    @functools.partial(
        pl.kernel,
        out_type=jax.ShapeDtypeStruct((rows, 64), F32),
        mesh=_mesh(),
        compiler_params=pltpu.CompilerParams(use_tc_tiling_on_sc=False),
        scratch_types=[
            pltpu.VMEM((pt, 128), I32),
            pltpu.VMEM((pt, 128), I32),
        ] + [pltpu.VMEM((128, 64), F32)] * 8
          + [pltpu.SemaphoreType.DMA] * 8,
    )
    def k(a_h, b_h, row_h, col_h, g_h, ridx, cidx,
          a0, a1, a2, a3, b0, b1, b2, b3,
          sg0, sg1, sg2, sg3, sw0, sw1, sw2, sw3):
        ba = (a0, a1, a2, a3)
        bb = (b0, b1, b2, b3)
        sg = (sg0, sg1, sg2, sg3)
        sw = (sw0, sw1, sw2, sw3)
        c = lax.axis_index("c")
        s = lax.axis_index("s")
        wid = c * 16 + s
        pltpu.sync_copy(row_h.at[pl.ds(wid * pt, pt)], ridx)
        pltpu.sync_copy(col_h.at[pl.ds(wid * pt, pt)], cidx)

        def fire_g(j, t):
            pltpu.async_copy(a_h.at[ridx.at[j]], ba[t], sg[t])
            pltpu.async_copy(b_h.at[cidx.at[j]], bb[t], sg[t])

        def wait_g(j, t):
            pltpu.make_async_copy(a_h.at[ridx.at[j]], ba[t], sg[t]).wait()
            pltpu.make_async_copy(b_h.at[cidx.at[j]], bb[t], sg[t]).wait()

        def add_bufs(t):
            def rbody(i, carry):
                for q in range(4):
                    sl_ = pl.ds(q * 16, 16)
                    ba[t][i, sl_] = ba[t][i, sl_] + bb[t][i, sl_]
                return carry

            lax.fori_loop(0, 128, rbody, 0)

        def fire_w(j, t):
            base = pl.multiple_of((wid * pt + j) * 128, 128)
            pltpu.async_copy(ba[t], g_h.at[pl.ds(base, 128)], sw[t])

        def wait_w(j, t):
            base = pl.multiple_of((wid * pt + j) * 128, 128)
            pltpu.make_async_copy(ba[t], g_h.at[pl.ds(base, 128)],
                                  sw[t]).wait()

        fire_g(0, 0)
        fire_g(1, 1)
        fire_g(2, 2)
        wait_g(0, 0)
        add_bufs(0)
        fire_w(0, 0)
        fire_g(3, 3)
        wait_g(1, 1)
        add_bufs(1)
        fire_w(1, 1)

        def body(kk, carry):
            j0 = kk * 4
            for t in range(4):
                j = j0 + t
                wait_w(j - 4, t)
                fire_g(j, t)
                wait_g(j - 2, (t + 2) % 4)
                add_bufs((t + 2) % 4)
                fire_w(j - 2, (t + 2) % 4)
            return carry

        lax.fori_loop(1, pt // 4, body, 0)
        wait_g(pt - 2, (pt - 2) % 4)
        add_bufs((pt - 2) % 4)
        fire_w(pt - 2, (pt - 2) % 4)
        wait_g(pt - 1, (pt - 1) % 4)
        add_bufs((pt - 1) % 4)
        fire_w(pt - 1, (pt - 1) % 4)
        for j in (pt - 4, pt - 3, pt - 2, pt - 1):
            wait_w(j, j % 4)

    return k(a, b, rowp, colp)


def _sc_scatter_edges(ee, colp, zeros_e, btcp=None, ones=None,
                      zc_e=None, zc_n3=None):
    pt = ECH // 32
    rpt = NP // 16
    counts = btcp is not None
    npt = (G * NCH) // 32 if counts else 0
    nrpt = (G * GACC) // 16

    outs = [jax.ShapeDtypeStruct((2, G * NP, 64), F32)]
    scr = ([pltpu.VMEM((G * pt, 128), I32)]
           + [pltpu.VMEM((128, 64), F32)] * 4
           + [pltpu.SemaphoreType.DMA] * 8)
    if counts:
        outs += [jax.ShapeDtypeStruct((2, G * NP, 16), F32),
                 jax.ShapeDtypeStruct((2, G * GACC, 16), F32)]
        scr += [pltpu.VMEM((npt, 128), I32),
                pltpu.VMEM((128, 16), F32),
                pltpu.VMEM_SHARED((NP, 16), F32),
                pltpu.VMEM_SHARED((G * GACC, 16), F32),
                pltpu.SemaphoreType.DMA]
    scr += [pltpu.VMEM_SHARED((NP, 64), F32)]

    @functools.partial(
        pl.kernel,
        out_type=tuple(outs) if counts else outs[0],
        mesh=_mesh(),
        compiler_params=pltpu.CompilerParams(use_tc_tiling_on_sc=False),
        scratch_types=scr,
    )
    def k(*refs):
        if counts:
            (d_h, i_h, z_h, b_h, o_h, ze_h, zn_h, out_h, ec_h, nc_h,
             iidx, d0, d1, d2, d3, sl0, sl1, sl2, sl3, ss0, ss1, ss2, ss3,
             nidx, obuf, eacc, nacc, scnt, acc) = refs
        else:
            (d_h, i_h, z_h, out_h,
             iidx, d0, d1, d2, d3, sl0, sl1, sl2, sl3, ss0, ss1, ss2, ss3,
             acc) = refs
        dbuf = (d0, d1, d2, d3)
        sl = (sl0, sl1, sl2, sl3)
        ss = (ss0, ss1, ss2, ss3)
        c = lax.axis_index("c")
        s = lax.axis_index("s")
        for g in range(G):
            pltpu.sync_copy(
                i_h.at[pl.ds(g * ECH + c * (ECH // 2) + s * pt, pt)],
                iidx.at[pl.ds(g * pt, pt)])
        if counts:
            nstart = c * ((G * NCH) // 2) + s * npt
            pltpu.sync_copy(b_h.at[pl.ds(nstart, npt)], nidx)
            pltpu.sync_copy(o_h, obuf)
            pltpu.sync_copy(zn_h.at[pl.ds(s * nrpt, nrpt)],
                            nacc.at[pl.ds(s * nrpt, nrpt)])

        for g in range(G):
            gbase = g * ECH + c * (ECH // 2) + s * pt

            def fire_l(j, t):
                base = pl.multiple_of((gbase + j) * 128, 128)
                pltpu.async_copy(d_h.at[pl.ds(base, 128)], dbuf[t], sl[t])

            def wait_l(j, t):
                base = pl.multiple_of((gbase + j) * 128, 128)
                pltpu.make_async_copy(d_h.at[pl.ds(base, 128)], dbuf[t],
                                      sl[t]).wait()

            def fire_s(j, t):
                pltpu.async_copy(dbuf[t], acc.at[iidx.at[g * pt + j]],
                                 ss[t], add=True)
                if counts:
                    pltpu.async_copy(obuf, eacc.at[iidx.at[g * pt + j]],
                                     scnt, add=True)

            def wait_s(j, t):
                pltpu.make_async_copy(dbuf[t], acc.at[iidx.at[g * pt + j]],
                                      ss[t]).wait()

            fire_l(0, 0)
            fire_l(1, 1)
            pltpu.sync_copy(z_h.at[pl.ds(s * rpt, rpt)],
                            acc.at[pl.ds(s * rpt, rpt)])
            if counts:
                pltpu.sync_copy(ze_h.at[pl.ds(s * rpt, rpt)],
                                eacc.at[pl.ds(s * rpt, rpt)])
            fire_l(2, 2)
            plsc.subcore_barrier()
            if counts and g == 0:
                for j in range(npt):
                    pltpu.async_copy(obuf, nacc.at[nidx.at[j]], scnt,
                                     add=True)
            wait_l(0, 0)
            fire_s(0, 0)
            fire_l(3, 3)
            wait_l(1, 1)
            fire_s(1, 1)

            def body(kk, carry):
                j0 = kk * 4
                for t in range(4):
                    j = j0 + t
                    wait_s(j - 4, t)
                    fire_l(j, t)
                    wait_l(j - 2, (t + 2) % 4)
                    fire_s(j - 2, (t + 2) % 4)
                return carry

            lax.fori_loop(1, pt // 4, body, 0)
            wait_l(pt - 2, (pt - 2) % 4)
            fire_s(pt - 2, (pt - 2) % 4)
            wait_l(pt - 1, (pt - 1) % 4)
            fire_s(pt - 1, (pt - 1) % 4)
            for j in (pt - 4, pt - 3, pt - 2, pt - 1):
                wait_s(j, j % 4)
            if counts:
                def drain(j, carry):
                    pltpu.make_async_copy(
                        obuf, eacc.at[iidx.at[g * pt + j]], scnt).wait()
                    return carry

                lax.fori_loop(g * pt, (g + 1) * pt, drain, 0)
                if g == 0:
                    for j in range(npt):
                        pltpu.make_async_copy(obuf, nacc.at[nidx.at[j]],
                                              scnt).wait()
            plsc.subcore_barrier()
            pltpu.sync_copy(acc.at[pl.ds(s * rpt, rpt)],
                            out_h.at[c, pl.ds(g * NP + s * rpt, rpt)])
            if counts:
                pltpu.sync_copy(eacc.at[pl.ds(s * rpt, rpt)],
                                ec_h.at[c, pl.ds(g * NP + s * rpt, rpt)])
                if g == 0:
                    pltpu.sync_copy(nacc.at[pl.ds(s * nrpt, nrpt)],
                                    nc_h.at[c, pl.ds(s * nrpt, nrpt)])
            if g + 1 < G:
                plsc.subcore_barrier()

    if counts:
        return k(ee, colp, zeros_e, btcp, ones, zc_e, zc_n3)
    return k(ee, colp, zeros_e)


def _sc_scatter_nodes(x2, btcp, zeros_n3):
    tch = btcp.shape[0]
    pt = tch // 32
    nrpt = (G * GACC) // 16

    @functools.partial(
        pl.kernel,
        out_type=jax.ShapeDtypeStruct((2, G * GACC, 128), F32),
        mesh=_mesh(),
        compiler_params=pltpu.CompilerParams(use_tc_tiling_on_sc=False),
        scratch_types=[pltpu.VMEM((pt, 128), I32)]
        + [pltpu.VMEM((128, 128), F32)] * 4
        + [pltpu.SemaphoreType.DMA] * 8
        + [pltpu.VMEM_SHARED((G * GACC, 128), F32)],
    )
    def k(d_h, i_h, z_h, out_h, iidx, d0, d1, d2, d3,
          sl0, sl1, sl2, sl3, ss0, ss1, ss2, ss3, acc):
        dbuf = (d0, d1, d2, d3)
        sl = (sl0, sl1, sl2, sl3)
        ss = (ss0, ss1, ss2, ss3)
        c = lax.axis_index("c")
        s = lax.axis_index("s")
        start = c * (tch // 2) + s * pt
        pltpu.sync_copy(i_h.at[pl.ds(start, pt)], iidx)

        def fire_l(j, t):
            base = pl.multiple_of((start + j) * 128, 128)
            pltpu.async_copy(d_h.at[pl.ds(base, 128)], dbuf[t], sl[t])

        def wait_l(j, t):
            base = pl.multiple_of((start + j) * 128, 128)
            pltpu.make_async_copy(d_h.at[pl.ds(base, 128)], dbuf[t],
                                  sl[t]).wait()

        def fire_s(j, t):
            pltpu.async_copy(dbuf[t], acc.at[iidx.at[j]], ss[t], add=True)

        def wait_s(j, t):
            pltpu.make_async_copy(dbuf[t], acc.at[iidx.at[j]], ss[t]).wait()

        fire_l(0, 0)
        fire_l(1, 1)
        fire_l(2, 2)
        pltpu.sync_copy(z_h.at[pl.ds(s * nrpt, nrpt)],
                        acc.at[pl.ds(s * nrpt, nrpt)])
        plsc.subcore_barrier()
        wait_l(0, 0)
        fire_s(0, 0)
        fire_l(3, 3)
        wait_l(1, 1)
        fire_s(1, 1)
        for j in range(4, pt):
            t = j % 4
            wait_s(j - 4, t)
            fire_l(j, t)
            wait_l(j - 2, (j - 2) % 4)
            fire_s(j - 2, (j - 2) % 4)
        wait_l(pt - 2, (pt - 2) % 4)
        fire_s(pt - 2, (pt - 2) % 4)
        wait_l(pt - 1, (pt - 1) % 4)
        fire_s(pt - 1, (pt - 1) % 4)
        for j in range(pt - 4, pt):
            wait_s(j, j % 4)
        plsc.subcore_barrier()
        pltpu.sync_copy(acc.at[pl.ds(s * nrpt, nrpt)],
                        out_h.at[c, pl.ds(s * nrpt, nrpt)])

    return k(x2, btcp, zeros_n3)


def _dot(x, w):
    return jnp.dot(x, w, preferred_element_type=F32)


def _tc_enc(x, w1, b1, w2, b2, wa, wb):
    n = x.shape[0]

    def body(x_r, w1_r, b1_r, w2_r, b2_r, wa_r, wb_r, ex_r, a_r, b_r):
        h = jnp.maximum(_dot(x_r[...], w1_r[...]) + b1_r[...], 0.0)
        e = _dot(h, w2_r[...]) + b2_r[...]
        ex_r[...] = e
        a_r[...] = _dot(e, wa_r[...])
        b_r[...] = _dot(e, wb_r[...])

    return pl.pallas_call(
        body, grid=(n // BLK,),
        in_specs=[_rows(BLK, 128), _full((128, 128)), _full((1, 128)),
                  _full((128, 128)), _full((1, 128)),
                  _full((128, 64)), _full((128, 64))],
        out_specs=[_rows(BLK, 128), _rows(BLK, 64), _rows(BLK, 64)],
        out_shape=[jax.ShapeDtypeStruct((n, 128), F32),
                   jax.ShapeDtypeStruct((n, 64), F32),
                   jax.ShapeDtypeStruct((n, 64), F32)],
    )(x, w1, b1, w2, b2, wa, wb)


def _tc_ee(ea8, w1p, b1, w2, b2):
    n = ea8.shape[0]

    def body(e_r, w1_r, b1_r, w2_r, b2_r, o_r):
        h = jnp.maximum(_dot(e_r[...], w1_r[...]) + b1_r[...], 0.0)
        o_r[...] = _dot(h, w2_r[...]) + b2_r[...]

    return pl.pallas_call(
        body, grid=(n // BLK,),
        in_specs=[_rows(BLK, 8), _full((8, 64)), _full((1, 64)),
                  _full((64, 64)), _full((1, 64))],
        out_specs=_rows(BLK, 64),
        out_shape=jax.ShapeDtypeStruct((n, 64), F32),
    )(ea8, w1p, b1, w2, b2)


def _tc_edge(gsum, ee, wc, b1, w2, b2):
    n = gsum.shape[0]

    def body(g_r, e_r, wc_r, b1_r, w2_r, b2_r, o_r):
        h = g_r[...] + _dot(e_r[...], wc_r[...]) + b1_r[...]
        h = jnp.maximum(h, 0.0)
        o_r[...] = _dot(h, w2_r[...]) + b2_r[...]

    return pl.pallas_call(
        body, grid=(n // BLK,),
        in_specs=[_rows(BLK, 64), _rows(BLK, 64),
                  _full((64, 64)), _full((1, 64)),
                  _full((64, 64)), _full((1, 64))],
        out_specs=_rows(BLK, 64),
        out_shape=jax.ShapeDtypeStruct((n, 64), F32),
    )(gsum, ee, wc, b1, w2, b2)


def _tc_edge0(gsum, ea8, we1p, be1, we2, be2, wc, b1, w2, b2):
    n = gsum.shape[0]

    def body(g_r, e_r, we1_r, be1_r, we2_r, be2_r, wc_r, b1_r, w2_r, b2_r,
             o_r):
        eh = jnp.maximum(_dot(e_r[...], we1_r[...]) + be1_r[...], 0.0)
        ee = _dot(eh, we2_r[...]) + be2_r[...]
        h = g_r[...] + _dot(ee, wc_r[...]) + b1_r[...]
        h = jnp.maximum(h, 0.0)
        o_r[...] = _dot(h, w2_r[...]) + b2_r[...]

    return pl.pallas_call(
        body, grid=(n // BLK,),
        in_specs=[_rows(BLK, 64), _rows(BLK, 8),
                  _full((8, 64)), _full((1, 64)),
                  _full((64, 64)), _full((1, 64)),
                  _full((64, 64)), _full((1, 64)),
                  _full((64, 64)), _full((1, 64))],
        out_specs=_rows(BLK, 64),
        out_shape=jax.ShapeDtypeStruct((n, 64), F32),
    )(gsum, ea8, we1p, be1, we2, be2, wc, b1, w2, b2)


def _tc_node(ex, esum, ecnt, wx, wagg, b1, w2, b2, wa, wb):
    n = ex.shape[0]

    def body(x_r, s0_r, s1_r, c0_r, c1_r, wx_r, wg_r, b1_r, w2_r, b2_r,
             wa_r, wb_r, x2_r, a_r, b_r):
        cnt = c0_r[0][:, 0:1] + c1_r[0][:, 0:1]
        rcp = 1.0 / jnp.maximum(cnt, 1.0)
        agg = (s0_r[0] + s1_r[0]) * rcp
        h = jnp.maximum(_dot(x_r[...], wx_r[...]) + _dot(agg, wg_r[...])
                        + b1_r[...], 0.0)
        x2 = _dot(h, w2_r[...]) + b2_r[...]
        x2_r[...] = x2
        a_r[...] = _dot(x2, wa_r[...])
        b_r[...] = _dot(x2, wb_r[...])

    return pl.pallas_call(
        body, grid=(n // BLK,),
        in_specs=[_rows(BLK, 128),
                  pl.BlockSpec((1, BLK, 64), lambda i: (0, i, 0)),
                  pl.BlockSpec((1, BLK, 64), lambda i: (1, i, 0)),
                  pl.BlockSpec((1, BLK, 16), lambda i: (0, i, 0)),
                  pl.BlockSpec((1, BLK, 16), lambda i: (1, i, 0)),
                  _full((128, 128)), _full((64, 128)), _full((1, 128)),
                  _full((128, 128)), _full((1, 128)),
                  _full((128, 64)), _full((128, 64))],
        out_specs=[_rows(BLK, 128), _rows(BLK, 64), _rows(BLK, 64)],
        out_shape=[jax.ShapeDtypeStruct((n, 128), F32),
                   jax.ShapeDtypeStruct((n, 64), F32),
                   jax.ShapeDtypeStruct((n, 64), F32)],
    )(ex, esum, esum, ecnt, ecnt, wx, wagg, b1, w2, b2, wa, wb)


def _tc_glob(u, nsum, ncnt, wu, wn, b1, w2, b2):
    def body(u_r, s0_r, s1_r, c0_r, c1_r, wu_r, wn_r, b1_r, w2_r, b2_r,
             o_r):
        cnt = c0_r[0][:, 0:1] + c1_r[0][:, 0:1]
        rcp = 1.0 / jnp.maximum(cnt, 1.0)
        nmean = (s0_r[0] + s1_r[0]) * rcp
        h = jnp.maximum(_dot(u_r[...], wu_r[...]) + _dot(nmean, wn_r[...])
                        + b1_r[...], 0.0)
        o_r[...] = _dot(h, w2_r[...]) + b2_r[...]

    return pl.pallas_call(
        body, grid=(G,),
        in_specs=[_rows(B, 64),
                  pl.BlockSpec((1, B, 128), lambda i: (0, 2 * i, 0)),
                  pl.BlockSpec((1, B, 128), lambda i: (1, 2 * i, 0)),
                  pl.BlockSpec((1, B, 16), lambda i: (0, 2 * i, 0)),
                  pl.BlockSpec((1, B, 16), lambda i: (1, 2 * i, 0)),
                  _full((64, 64)), _full((128, 64)), _full((1, 64)),
                  _full((64, 64)), _full((1, 64))],
        out_specs=_rows(B, 64),
        out_shape=jax.ShapeDtypeStruct((G * B, 64), F32),
    )(u, nsum, nsum, ncnt, ncnt, wu, wn, b1, w2, b2)


def _tc_phase2(u0, nsum, ncnt, wu, wn, bg1, wg2, bg2, scal4,
               wn21, bn21, wn22, bn22,
               we21, be21, we22, be22,
               wE1, bE1, wE2, bE2,
               wX1, bX1, wX2, bX2,
               wG1, bG1, wG2, bG2,
               wl1, bl1, wl2, bl2):
    def body(u0_r, ns_r, nc_r, wu_r, wn_r, bg1_r, wg2_r, bg2_r, sc_r,
             wn21_r, bn21_r, wn22_r, bn22_r,
             we21_r, be21_r, we22_r, be22_r,
             wE1_r, bE1_r, wE2_r, bE2_r,
             wX1_r, bX1_r, wX2_r, bX2_r,
             wG1_r, bG1_r, wG2_r, bG2_r,
             wl1_r, bl1_r, wl2_r, bl2_r, o_r):
        u0v = u0_r[...]
        nsv = ns_r[...]
        ncv = nc_r[...]
        uf = []
        for g in range(G):
            r0, r1 = g * GACC, g * GACC + B
            cnt = ncv[0, r0:r1, 0:1] + ncv[1, r0:r1, 0:1]
            rcp = 1.0 / jnp.maximum(cnt, 1.0)
            nmean = (nsv[0, r0:r1] + nsv[1, r0:r1]) * rcp
            hu = jnp.maximum(_dot(u0v[g * B:(g + 1) * B], wu_r[...])
                             + _dot(nmean, wn_r[...]) + bg1_r[...], 0.0)
            uf.append(_dot(hu, wg2_r[...]) + bg2_r[...])
        u1_v, u2_v, um_v = uf
        sc = sc_r[...]
        t = sc[:, 0:1]
        r1 = sc[:, 1:2] / sc[:, 3:4]
        r2 = sc[:, 2:3] / sc[:, 3:4]
        we21 = we21_r[...]
        h1 = jnp.maximum(t * we21[0:1, :] + r1 * we21[1:2, :] + be21_r[...],
                         0.0)
        c1 = _dot(h1, we22_r[...]) + be22_r[...]
        h2 = jnp.maximum(t * we21[0:1, :] + r2 * we21[1:2, :] + be21_r[...],
                         0.0)
        c2 = _dot(h2, we22_r[...]) + be22_r[...]

        def enc(u):
            h = jnp.maximum(_dot(u, wn21_r[...]) + bn21_r[...], 0.0)
            return _dot(h, wn22_r[...]) + bn22_r[...]

        n0 = enc(u1_v)
        n1 = enc(u2_v)
        n2 = enc(um_v)
        a0 = a1 = c1
        a2 = a3 = c2
        ug = jnp.full((B, 32), 0.1, F32)
        wE1v = wE1_r[...]
        ws, wd, wf = wE1v[0:64], wE1v[64:128], wE1v[128:192]
        wX1v = wX1_r[...]
        wxn, wan = wX1v[0:64], wX1v[64:128]
        wG1v = wG1_r[...]
        wug, wng = wG1v[0:32], wG1v[32:96]
        for _ in range(2):
            def edg(sv, dv, av):
                h = jnp.maximum(_dot(sv, ws) + _dot(dv, wd) + _dot(av, wf)
                                + bE1_r[...], 0.0)
                return _dot(h, wE2_r[...]) + bE2_r[...]

            e0 = edg(n0, n2, a0)
            e1 = edg(n2, n0, a1)
            e2 = edg(n1, n2, a2)
            e3 = edg(n2, n1, a3)

            def nod(v, ag):
                h = jnp.maximum(_dot(v, wxn) + _dot(ag, wan) + bX1_r[...],
                                0.0)
                return _dot(h, wX2_r[...]) + bX2_r[...]

            n0 = nod(n0, e1)
            n1 = nod(n1, e3)
            n2 = nod(n2, (e0 + e2) * 0.5)
            nm = (n0 + n1 + n2) / 3.0
            hg = jnp.maximum(_dot(ug, wug) + _dot(nm, wng) + bG1_r[...], 0.0)
            ug = _dot(hg, wG2_r[...]) + bG2_r[...]
            a0, a1, a2, a3 = e0, e1, e2, e3
        hl = jnp.maximum(_dot(ug, wl1_r[...]) + bl1_r[...], 0.0)
        o_r[...] = _dot(hl, wl2_r[...]) + bl2_r[...]

    return pl.pallas_call(
        body,
        out_shape=jax.ShapeDtypeStruct((B, 15), F32),
    )(u0, nsum, ncnt, wu, wn, bg1, wg2, bg2, scal4,
      wn21, bn21, wn22, bn22,
      we21, be21, we22, be22,
      wE1, bE1, wE2, bE2,
      wX1, bX1, wX2, bX2,
      wG1, bG1, wG2, bG2,
      wl1, bl1, wl2, bl2)


def _row2(v):
    return v.reshape(1, -1)


def kernel(x_p1, ei_p1, ea_p1, y_p1, btc_p1, x_p2, ei_p2, ea_p2, y_p2,
           btc_p2, x_pm, ei_pm, ea_pm, y_pm, btc_pm, Temperature, params):
    p = params
    (we1, be1), (we2, be2) = p['enc_node_1']
    (wee1, bee1), (wee2, bee2) = p['enc_edge_1']
    (wE1, bE1), (wE2, bE2) = p['edge1']
    (wN1, bN1), (wN2, bN2) = p['node1']
    (wG1, bG1), (wG2, bG2) = p['glob1']
    wa, wb, wc = wE1[0:128], wE1[128:256], wE1[256:320]
    wx, wagg = wN1[0:128], wN1[128:192]
    wu, wn = wG1[0:64], wG1[64:192]
    wee1p = jnp.concatenate([wee1, jnp.zeros((4, 64), F32)], axis=0)

    zeros_e = jnp.zeros((NP, 64), F32)
    zeros_n3 = jnp.zeros((G * GACC, 128), F32)
    zc_e = jnp.zeros((NP, 16), F32)
    zc_n3 = jnp.zeros((G * GACC, 16), F32)
    ones16 = jnp.ones((128, 16), F32)

    graphs = ((x_p1, ei_p1, ea_p1, btc_p1),
              (x_p2, ei_p2, ea_p2, btc_p2),
              (x_pm, ei_pm, ea_pm, btc_pm))
    rowo, colo, colu, btco, xps, ea8s = [], [], [], [], [], []
    for g, (x, ei, ea, btc) in enumerate(graphs):
        pad_e = jnp.full((EP - NE,), NN, I32)
        rowo.append(jnp.concatenate([ei[0], pad_e]) + g * NP)
        colo.append(jnp.concatenate([ei[1], pad_e]) + g * NP)
        colu.append(jnp.concatenate([ei[1], pad_e]))
        btco.append(jnp.concatenate(
            [btc, jnp.full((NP - NN,), B, I32)]) + g * GACC)
        xps.append(jnp.pad(x, ((0, NP - NN), (0, 0))))
        ea8s.append(jnp.pad(ea, ((0, EP - NE), (0, 4))))
    rowp = jnp.concatenate(rowo).reshape(G * ECH, 128)
    colp = jnp.concatenate(colo).reshape(G * ECH, 128)
    colpu = jnp.concatenate(colu).reshape(G * ECH, 128)
    btcp = jnp.concatenate(btco).reshape(G * NCH, 128)
    xall = jnp.concatenate(xps)
    ea8all = jnp.concatenate(ea8s)

    ex, a, b = _tc_enc(xall, we1, _row2(be1), we2, _row2(be2), wa, wb)
    u = jnp.full((G * B, 64), 0.1, F32)
    for r in range(2):
        gsum = _sc_gather(a, b, rowp, colp)
        if r == 0:
            ee = _tc_edge0(gsum, ea8all, wee1p, _row2(bee1), wee2,
                           _row2(bee2), wc, _row2(bE1), wE2, _row2(bE2))
            esum, ecnt, ncnt = _sc_scatter_edges(
                ee, colpu, zeros_e, btcp, ones16, zc_e, zc_n3)
        else:
            ee = _tc_edge(gsum, ee, wc, _row2(bE1), wE2, _row2(bE2))
            esum = _sc_scatter_edges(ee, colpu, zeros_e)
        ex, a, b = _tc_node(ex, esum, ecnt, wx, wagg, _row2(bN1), wN2,
                            _row2(bN2), wa, wb)
        nsum = _sc_scatter_nodes(ex, btcp, zeros_n3)
        if r == 0:
            u = _tc_glob(u, nsum, ncnt, wu, wn, _row2(bG1), wG2,
                         _row2(bG2))

    (wn21, bn21), (wn22, bn22) = p['enc_node_2']
    (we21, be21), (we22, be22) = p['enc_edge_2']
    (wE21, bE21), (wE22, bE22) = p['edge2']
    (wX21, bX21), (wX22, bX22) = p['node2']
    (wG21, bG21), (wG22, bG22) = p['glob2']
    (wl1, bl1), (wl2, bl2) = p['last']
    scal4 = jnp.stack([Temperature, y_p1, y_p2, y_pm], axis=1)
    return _tc_phase2(u, nsum, ncnt, wu, wn, _row2(bG1), wG2, _row2(bG2),
                      scal4,
                      wn21, _row2(bn21), wn22, _row2(bn22),
                      we21, _row2(be21), we22, _row2(be22),
                      wE21, _row2(bE21), wE22, _row2(bE22),
                      wX21, _row2(bX21), wX22, _row2(bX22),
                      wG21, _row2(bG21), wG22, _row2(bG22),
                      wl1, _row2(bl1), wl2, _row2(bl2))


# speedup vs baseline: 1.1678x; 1.0650x over previous
import functools

import jax
import jax.numpy as jnp
from jax import lax
from jax.experimental import pallas as pl
from jax.experimental.pallas import tpu as pltpu
from jax.experimental.pallas import tpu_sc as plsc

F32 = jnp.float32
BF16 = jnp.bfloat16
I32 = jnp.int32

NN = 10000
NE = 160000
B = 256
NP = 12288
EP = 163840
G = 3
GACC = 512
ECH = EP // 128
NCH = NP // 128
BLK = 4096


def _rows(blk, w):
    return pl.BlockSpec((blk, w), lambda i: (i, 0))


def _full(shape):
    return pl.BlockSpec(shape, lambda i: tuple(0 for _ in shape))


def _mesh():
    return plsc.VectorSubcoreMesh(core_axis_name="c", subcore_axis_name="s")


def _sc_gather(a, b, rowp, colp):
    tch = rowp.shape[0]
    pt = tch // 32
    rows = tch * 128

    @functools.partial(
        pl.kernel,
        out_type=jax.ShapeDtypeStruct((rows, 64), F32),
        mesh=_mesh(),
        compiler_params=pltpu.CompilerParams(use_tc_tiling_on_sc=False),
        scratch_types=[
            pltpu.VMEM((pt, 128), I32),
            pltpu.VMEM((pt, 128), I32),
        ] + [pltpu.VMEM((128, 64), F32)] * 8
          + [pltpu.SemaphoreType.DMA] * 8,
    )
    def k(a_h, b_h, row_h, col_h, g_h, ridx, cidx,
          a0, a1, a2, a3, b0, b1, b2, b3,
          sg0, sg1, sg2, sg3, sw0, sw1, sw2, sw3):
        ba = (a0, a1, a2, a3)
        bb = (b0, b1, b2, b3)
        sg = (sg0, sg1, sg2, sg3)
        sw = (sw0, sw1, sw2, sw3)
        c = lax.axis_index("c")
        s = lax.axis_index("s")
        wid = c * 16 + s
        pltpu.sync_copy(row_h.at[pl.ds(wid * pt, pt)], ridx)
        pltpu.sync_copy(col_h.at[pl.ds(wid * pt, pt)], cidx)

        def fire_g(j, t):
            pltpu.async_copy(a_h.at[ridx.at[j]], ba[t], sg[t])
            pltpu.async_copy(b_h.at[cidx.at[j]], bb[t], sg[t])

        def wait_g(j, t):
            pltpu.make_async_copy(a_h.at[ridx.at[j]], ba[t], sg[t]).wait()
            pltpu.make_async_copy(b_h.at[cidx.at[j]], bb[t], sg[t]).wait()

        def add_bufs(t):
            def rbody(i, carry):
                for q in range(4):
                    sl_ = pl.ds(q * 16, 16)
                    ba[t][i, sl_] = ba[t][i, sl_] + bb[t][i, sl_]
                return carry

            lax.fori_loop(0, 128, rbody, 0)

        def fire_w(j, t):
            base = pl.multiple_of((wid * pt + j) * 128, 128)
            pltpu.async_copy(ba[t], g_h.at[pl.ds(base, 128)], sw[t])

        def wait_w(j, t):
            base = pl.multiple_of((wid * pt + j) * 128, 128)
            pltpu.make_async_copy(ba[t], g_h.at[pl.ds(base, 128)],
                                  sw[t]).wait()

        fire_g(0, 0)
        fire_g(1, 1)
        fire_g(2, 2)
        wait_g(0, 0)
        add_bufs(0)
        fire_w(0, 0)
        fire_g(3, 3)
        wait_g(1, 1)
        add_bufs(1)
        fire_w(1, 1)

        def body(kk, carry):
            j0 = kk * 4
            for t in range(4):
                j = j0 + t
                wait_w(j - 4, t)
                fire_g(j, t)
                wait_g(j - 2, (t + 2) % 4)
                add_bufs((t + 2) % 4)
                fire_w(j - 2, (t + 2) % 4)
            return carry

        lax.fori_loop(1, pt // 4, body, 0)
        wait_g(pt - 2, (pt - 2) % 4)
        add_bufs((pt - 2) % 4)
        fire_w(pt - 2, (pt - 2) % 4)
        wait_g(pt - 1, (pt - 1) % 4)
        add_bufs((pt - 1) % 4)
        fire_w(pt - 1, (pt - 1) % 4)
        for j in (pt - 4, pt - 3, pt - 2, pt - 1):
            wait_w(j, j % 4)

    return k(a, b, rowp, colp)


def _sc_scatter_edges(ee, colp, zeros_e, btcp=None, ones=None,
                      zc_e=None, zc_n3=None):
    pt = ECH // 32
    rpt = NP // 16
    counts = btcp is not None
    npt = (G * NCH) // 32 if counts else 0
    nrpt = (G * GACC) // 16

    outs = [jax.ShapeDtypeStruct((2, G * NP, 64), F32)]
    scr = ([pltpu.VMEM((G * pt, 128), I32)]
           + [pltpu.VMEM((128, 64), F32)] * 4
           + [pltpu.SemaphoreType.DMA] * 8)
    if counts:
        outs += [jax.ShapeDtypeStruct((2, G * NP, 16), F32),
                 jax.ShapeDtypeStruct((2, G * GACC, 16), F32)]
        scr += [pltpu.VMEM((npt, 128), I32),
                pltpu.VMEM((128, 16), F32),
                pltpu.VMEM_SHARED((NP, 16), F32),
                pltpu.VMEM_SHARED((G * GACC, 16), F32),
                pltpu.SemaphoreType.DMA]
    scr += [pltpu.VMEM_SHARED((NP, 64), F32)]

    @functools.partial(
        pl.kernel,
        out_type=tuple(outs) if counts else outs[0],
        mesh=_mesh(),
        compiler_params=pltpu.CompilerParams(use_tc_tiling_on_sc=False),
        scratch_types=scr,
    )
    def k(*refs):
        if counts:
            (d_h, i_h, z_h, b_h, o_h, ze_h, zn_h, out_h, ec_h, nc_h,
             iidx, d0, d1, d2, d3, sl0, sl1, sl2, sl3, ss0, ss1, ss2, ss3,
             nidx, obuf, eacc, nacc, scnt, acc) = refs
        else:
            (d_h, i_h, z_h, out_h,
             iidx, d0, d1, d2, d3, sl0, sl1, sl2, sl3, ss0, ss1, ss2, ss3,
             acc) = refs
        dbuf = (d0, d1, d2, d3)
        sl = (sl0, sl1, sl2, sl3)
        ss = (ss0, ss1, ss2, ss3)
        c = lax.axis_index("c")
        s = lax.axis_index("s")
        for g in range(G):
            pltpu.sync_copy(
                i_h.at[pl.ds(g * ECH + c * (ECH // 2) + s * pt, pt)],
                iidx.at[pl.ds(g * pt, pt)])
        if counts:
            nstart = c * ((G * NCH) // 2) + s * npt
            pltpu.sync_copy(b_h.at[pl.ds(nstart, npt)], nidx)
            pltpu.sync_copy(o_h, obuf)
            pltpu.sync_copy(zn_h.at[pl.ds(s * nrpt, nrpt)],
                            nacc.at[pl.ds(s * nrpt, nrpt)])

        for g in range(G):
            gbase = g * ECH + c * (ECH // 2) + s * pt

            def fire_l(j, t):
                base = pl.multiple_of((gbase + j) * 128, 128)
                pltpu.async_copy(d_h.at[pl.ds(base, 128)], dbuf[t], sl[t])

            def wait_l(j, t):
                base = pl.multiple_of((gbase + j) * 128, 128)
                pltpu.make_async_copy(d_h.at[pl.ds(base, 128)], dbuf[t],
                                      sl[t]).wait()

            def fire_s(j, t):
                pltpu.async_copy(dbuf[t], acc.at[iidx.at[g * pt + j]],
                                 ss[t], add=True)
                if counts:
                    pltpu.async_copy(obuf, eacc.at[iidx.at[g * pt + j]],
                                     scnt, add=True)

            def wait_s(j, t):
                pltpu.make_async_copy(dbuf[t], acc.at[iidx.at[g * pt + j]],
                                      ss[t]).wait()

            fire_l(0, 0)
            fire_l(1, 1)
            pltpu.sync_copy(z_h.at[pl.ds(s * rpt, rpt)],
                            acc.at[pl.ds(s * rpt, rpt)])
            if counts:
                pltpu.sync_copy(ze_h.at[pl.ds(s * rpt, rpt)],
                                eacc.at[pl.ds(s * rpt, rpt)])
            fire_l(2, 2)
            plsc.subcore_barrier()
            if counts and g == 0:
                for j in range(npt):
                    pltpu.async_copy(obuf, nacc.at[nidx.at[j]], scnt,
                                     add=True)
            wait_l(0, 0)
            fire_s(0, 0)
            fire_l(3, 3)
            wait_l(1, 1)
            fire_s(1, 1)

            def body(kk, carry):
                j0 = kk * 4
                for t in range(4):
                    j = j0 + t
                    wait_s(j - 4, t)
                    fire_l(j, t)
                    wait_l(j - 2, (t + 2) % 4)
                    fire_s(j - 2, (t + 2) % 4)
                return carry

            lax.fori_loop(1, pt // 4, body, 0)
            wait_l(pt - 2, (pt - 2) % 4)
            fire_s(pt - 2, (pt - 2) % 4)
            wait_l(pt - 1, (pt - 1) % 4)
            fire_s(pt - 1, (pt - 1) % 4)
            for j in (pt - 4, pt - 3, pt - 2, pt - 1):
                wait_s(j, j % 4)
            if counts:
                def drain(j, carry):
                    pltpu.make_async_copy(
                        obuf, eacc.at[iidx.at[g * pt + j]], scnt).wait()
                    return carry

                lax.fori_loop(g * pt, (g + 1) * pt, drain, 0)
                if g == 0:
                    for j in range(npt):
                        pltpu.make_async_copy(obuf, nacc.at[nidx.at[j]],
                                              scnt).wait()
            plsc.subcore_barrier()
            pltpu.sync_copy(acc.at[pl.ds(s * rpt, rpt)],
                            out_h.at[c, pl.ds(g * NP + s * rpt, rpt)])
            if counts:
                pltpu.sync_copy(eacc.at[pl.ds(s * rpt, rpt)],
                                ec_h.at[c, pl.ds(g * NP + s * rpt, rpt)])
                if g == 0:
                    pltpu.sync_copy(nacc.at[pl.ds(s * nrpt, nrpt)],
                                    nc_h.at[c, pl.ds(s * nrpt, nrpt)])
            if g + 1 < G:
                plsc.subcore_barrier()

    if counts:
        return k(ee, colp, zeros_e, btcp, ones, zc_e, zc_n3)
    return k(ee, colp, zeros_e)


def _sc_scatter_nodes(x2, btcp, zeros_n3):
    tch = btcp.shape[0]
    pt = tch // 32
    nrpt = (G * GACC) // 16

    @functools.partial(
        pl.kernel,
        out_type=jax.ShapeDtypeStruct((2, G * GACC, 128), F32),
        mesh=_mesh(),
        compiler_params=pltpu.CompilerParams(use_tc_tiling_on_sc=False),
        scratch_types=[pltpu.VMEM((pt, 128), I32)]
        + [pltpu.VMEM((128, 128), F32)] * 4
        + [pltpu.SemaphoreType.DMA] * 8
        + [pltpu.VMEM_SHARED((G * GACC, 128), F32)],
    )
    def k(d_h, i_h, z_h, out_h, iidx, d0, d1, d2, d3,
          sl0, sl1, sl2, sl3, ss0, ss1, ss2, ss3, acc):
        dbuf = (d0, d1, d2, d3)
        sl = (sl0, sl1, sl2, sl3)
        ss = (ss0, ss1, ss2, ss3)
        c = lax.axis_index("c")
        s = lax.axis_index("s")
        start = c * (tch // 2) + s * pt
        pltpu.sync_copy(i_h.at[pl.ds(start, pt)], iidx)

        def fire_l(j, t):
            base = pl.multiple_of((start + j) * 128, 128)
            pltpu.async_copy(d_h.at[pl.ds(base, 128)], dbuf[t], sl[t])

        def wait_l(j, t):
            base = pl.multiple_of((start + j) * 128, 128)
            pltpu.make_async_copy(d_h.at[pl.ds(base, 128)], dbuf[t],
                                  sl[t]).wait()

        def fire_s(j, t):
            pltpu.async_copy(dbuf[t], acc.at[iidx.at[j]], ss[t], add=True)

        def wait_s(j, t):
            pltpu.make_async_copy(dbuf[t], acc.at[iidx.at[j]], ss[t]).wait()

        fire_l(0, 0)
        fire_l(1, 1)
        fire_l(2, 2)
        pltpu.sync_copy(z_h.at[pl.ds(s * nrpt, nrpt)],
                        acc.at[pl.ds(s * nrpt, nrpt)])
        plsc.subcore_barrier()
        wait_l(0, 0)
        fire_s(0, 0)
        fire_l(3, 3)
        wait_l(1, 1)
        fire_s(1, 1)
        for j in range(4, pt):
            t = j % 4
            wait_s(j - 4, t)
            fire_l(j, t)
            wait_l(j - 2, (j - 2) % 4)
            fire_s(j - 2, (j - 2) % 4)
        wait_l(pt - 2, (pt - 2) % 4)
        fire_s(pt - 2, (pt - 2) % 4)
        wait_l(pt - 1, (pt - 1) % 4)
        fire_s(pt - 1, (pt - 1) % 4)
        for j in range(pt - 4, pt):
            wait_s(j, j % 4)
        plsc.subcore_barrier()
        pltpu.sync_copy(acc.at[pl.ds(s * nrpt, nrpt)],
                        out_h.at[c, pl.ds(s * nrpt, nrpt)])

    return k(x2, btcp, zeros_n3)


def _dot(x, w):
    return jnp.dot(x, w, preferred_element_type=F32)


def _tc_enc(x, w1, b1, w2, b2, wa, wb):
    n = x.shape[0]

    def body(x_r, w1_r, b1_r, w2_r, b2_r, wa_r, wb_r, ex_r, a_r, b_r):
        h = jnp.maximum(_dot(x_r[...], w1_r[...]) + b1_r[...], 0.0)
        e = _dot(h, w2_r[...]) + b2_r[...]
        ex_r[...] = e
        a_r[...] = _dot(e, wa_r[...])
        b_r[...] = _dot(e, wb_r[...])

    return pl.pallas_call(
        body, grid=(n // BLK,),
        in_specs=[_rows(BLK, 128), _full((128, 128)), _full((1, 128)),
                  _full((128, 128)), _full((1, 128)),
                  _full((128, 64)), _full((128, 64))],
        out_specs=[_rows(BLK, 128), _rows(BLK, 64), _rows(BLK, 64)],
        out_shape=[jax.ShapeDtypeStruct((n, 128), F32),
                   jax.ShapeDtypeStruct((n, 64), F32),
                   jax.ShapeDtypeStruct((n, 64), F32)],
    )(x, w1, b1, w2, b2, wa, wb)


def _tc_ee(ea8, w1p, b1, w2, b2):
    n = ea8.shape[0]

    def body(e_r, w1_r, b1_r, w2_r, b2_r, o_r):
        h = jnp.maximum(_dot(e_r[...], w1_r[...]) + b1_r[...], 0.0)
        o_r[...] = _dot(h, w2_r[...]) + b2_r[...]

    return pl.pallas_call(
        body, grid=(n // BLK,),
        in_specs=[_rows(BLK, 8), _full((8, 64)), _full((1, 64)),
                  _full((64, 64)), _full((1, 64))],
        out_specs=_rows(BLK, 64),
        out_shape=jax.ShapeDtypeStruct((n, 64), F32),
    )(ea8, w1p, b1, w2, b2)


def _tc_edge(gsum, ee, wc, b1, w2, b2):
    n = gsum.shape[0]

    def body(g_r, e_r, wc_r, b1_r, w2_r, b2_r, o_r):
        h = g_r[...] + _dot(e_r[...], wc_r[...]) + b1_r[...]
        h = jnp.maximum(h, 0.0)
        o_r[...] = _dot(h, w2_r[...]) + b2_r[...]

    return pl.pallas_call(
        body, grid=(n // BLK,),
        in_specs=[_rows(BLK, 64), _rows(BLK, 64),
                  _full((64, 64)), _full((1, 64)),
                  _full((64, 64)), _full((1, 64))],
        out_specs=_rows(BLK, 64),
        out_shape=jax.ShapeDtypeStruct((n, 64), F32),
    )(gsum, ee, wc, b1, w2, b2)


def _tc_edge0(gsum, ea8, we1p, be1, we2, be2, wc, b1, w2, b2):
    n = gsum.shape[0]

    def body(g_r, e_r, we1_r, be1_r, we2_r, be2_r, wc_r, b1_r, w2_r, b2_r,
             o_r):
        eh = jnp.maximum(_dot(e_r[...], we1_r[...]) + be1_r[...], 0.0)
        ee = _dot(eh, we2_r[...]) + be2_r[...]
        h = g_r[...] + _dot(ee, wc_r[...]) + b1_r[...]
        h = jnp.maximum(h, 0.0)
        o_r[...] = _dot(h, w2_r[...]) + b2_r[...]

    return pl.pallas_call(
        body, grid=(n // BLK,),
        in_specs=[_rows(BLK, 64), _rows(BLK, 8),
                  _full((8, 64)), _full((1, 64)),
                  _full((64, 64)), _full((1, 64)),
                  _full((64, 64)), _full((1, 64)),
                  _full((64, 64)), _full((1, 64))],
        out_specs=_rows(BLK, 64),
        out_shape=jax.ShapeDtypeStruct((n, 64), F32),
    )(gsum, ea8, we1p, be1, we2, be2, wc, b1, w2, b2)


def _tc_node(ex, esum, ecnt, wx, wagg, b1, w2, b2, wa, wb):
    n = ex.shape[0]

    def body(x_r, s0_r, s1_r, c0_r, c1_r, wx_r, wg_r, b1_r, w2_r, b2_r,
             wa_r, wb_r, x2_r, a_r, b_r):
        cnt = c0_r[0][:, 0:1] + c1_r[0][:, 0:1]
        rcp = 1.0 / jnp.maximum(cnt, 1.0)
        agg = (s0_r[0] + s1_r[0]) * rcp
        h = jnp.maximum(_dot(x_r[...], wx_r[...]) + _dot(agg, wg_r[...])
                        + b1_r[...], 0.0)
        x2 = _dot(h, w2_r[...]) + b2_r[...]
        x2_r[...] = x2
        a_r[...] = _dot(x2, wa_r[...])
        b_r[...] = _dot(x2, wb_r[...])

    return pl.pallas_call(
        body, grid=(n // BLK,),
        in_specs=[_rows(BLK, 128),
                  pl.BlockSpec((1, BLK, 64), lambda i: (0, i, 0)),
                  pl.BlockSpec((1, BLK, 64), lambda i: (1, i, 0)),
                  pl.BlockSpec((1, BLK, 16), lambda i: (0, i, 0)),
                  pl.BlockSpec((1, BLK, 16), lambda i: (1, i, 0)),
                  _full((128, 128)), _full((64, 128)), _full((1, 128)),
                  _full((128, 128)), _full((1, 128)),
                  _full((128, 64)), _full((128, 64))],
        out_specs=[_rows(BLK, 128), _rows(BLK, 64), _rows(BLK, 64)],
        out_shape=[jax.ShapeDtypeStruct((n, 128), F32),
                   jax.ShapeDtypeStruct((n, 64), F32),
                   jax.ShapeDtypeStruct((n, 64), F32)],
    )(ex, esum, esum, ecnt, ecnt, wx, wagg, b1, w2, b2, wa, wb)


def _tc_glob(u, nsum, ncnt, wu, wn, b1, w2, b2):
    def body(u_r, s0_r, s1_r, c0_r, c1_r, wu_r, wn_r, b1_r, w2_r, b2_r,
             o_r):
        cnt = c0_r[0][:, 0:1] + c1_r[0][:, 0:1]
        rcp = 1.0 / jnp.maximum(cnt, 1.0)
        nmean = (s0_r[0] + s1_r[0]) * rcp
        h = jnp.maximum(_dot(u_r[...], wu_r[...]) + _dot(nmean, wn_r[...])
                        + b1_r[...], 0.0)
        o_r[...] = _dot(h, w2_r[...]) + b2_r[...]

    return pl.pallas_call(
        body, grid=(G,),
        in_specs=[_rows(B, 64),
                  pl.BlockSpec((1, B, 128), lambda i: (0, 2 * i, 0)),
                  pl.BlockSpec((1, B, 128), lambda i: (1, 2 * i, 0)),
                  pl.BlockSpec((1, B, 16), lambda i: (0, 2 * i, 0)),
                  pl.BlockSpec((1, B, 16), lambda i: (1, 2 * i, 0)),
                  _full((64, 64)), _full((128, 64)), _full((1, 64)),
                  _full((64, 64)), _full((1, 64))],
        out_specs=_rows(B, 64),
        out_shape=jax.ShapeDtypeStruct((G * B, 64), F32),
    )(u, nsum, nsum, ncnt, ncnt, wu, wn, b1, w2, b2)


def _tc_phase2(u0, nsum, ncnt, wu, wn, bg1, wg2, bg2, scal4,
               wn21, bn21, wn22, bn22,
               we21, be21, we22, be22,
               wE1, bE1, wE2, bE2,
               wX1, bX1, wX2, bX2,
               wG1, bG1, wG2, bG2,
               wl1, bl1, wl2, bl2):
    def body(u0_r, ns_r, nc_r, wu_r, wn_r, bg1_r, wg2_r, bg2_r, sc_r,
             wn21_r, bn21_r, wn22_r, bn22_r,
             we21_r, be21_r, we22_r, be22_r,
             wE1_r, bE1_r, wE2_r, bE2_r,
             wX1_r, bX1_r, wX2_r, bX2_r,
             wG1_r, bG1_r, wG2_r, bG2_r,
             wl1_r, bl1_r, wl2_r, bl2_r, o_r):
        u0v = u0_r[...]
        nsv = ns_r[...]
        ncv = nc_r[...]
        uf = []
        for g in range(G):
            r0, r1 = g * GACC, g * GACC + B
            cnt = ncv[0, r0:r1, 0:1] + ncv[1, r0:r1, 0:1]
            rcp = 1.0 / jnp.maximum(cnt, 1.0)
            nmean = (nsv[0, r0:r1] + nsv[1, r0:r1]) * rcp
            hu = jnp.maximum(_dot(u0v[g * B:(g + 1) * B], wu_r[...])
                             + _dot(nmean, wn_r[...]) + bg1_r[...], 0.0)
            uf.append(_dot(hu, wg2_r[...]) + bg2_r[...])
        u1_v, u2_v, um_v = uf
        sc = sc_r[...]
        t = sc[:, 0:1]
        r1 = sc[:, 1:2] / sc[:, 3:4]
        r2 = sc[:, 2:3] / sc[:, 3:4]
        we21 = we21_r[...]
        h1 = jnp.maximum(t * we21[0:1, :] + r1 * we21[1:2, :] + be21_r[...],
                         0.0)
        c1 = _dot(h1, we22_r[...]) + be22_r[...]
        h2 = jnp.maximum(t * we21[0:1, :] + r2 * we21[1:2, :] + be21_r[...],
                         0.0)
        c2 = _dot(h2, we22_r[...]) + be22_r[...]

        def enc(u):
            h = jnp.maximum(_dot(u, wn21_r[...]) + bn21_r[...], 0.0)
            return _dot(h, wn22_r[...]) + bn22_r[...]

        n0 = enc(u1_v)
        n1 = enc(u2_v)
        n2 = enc(um_v)
        a0 = a1 = c1
        a2 = a3 = c2
        ug = jnp.full((B, 32), 0.1, F32)
        wE1v = wE1_r[...]
        ws, wd, wf = wE1v[0:64], wE1v[64:128], wE1v[128:192]
        wX1v = wX1_r[...]
        wxn, wan = wX1v[0:64], wX1v[64:128]
        wG1v = wG1_r[...]
        wug, wng = wG1v[0:32], wG1v[32:96]
        for _ in range(2):
            def edg(sv, dv, av):
                h = jnp.maximum(_dot(sv, ws) + _dot(dv, wd) + _dot(av, wf)
                                + bE1_r[...], 0.0)
                return _dot(h, wE2_r[...]) + bE2_r[...]

            e0 = edg(n0, n2, a0)
            e1 = edg(n2, n0, a1)
            e2 = edg(n1, n2, a2)
            e3 = edg(n2, n1, a3)

            def nod(v, ag):
                h = jnp.maximum(_dot(v, wxn) + _dot(ag, wan) + bX1_r[...],
                                0.0)
                return _dot(h, wX2_r[...]) + bX2_r[...]

            n0 = nod(n0, e1)
            n1 = nod(n1, e3)
            n2 = nod(n2, (e0 + e2) * 0.5)
            nm = (n0 + n1 + n2) / 3.0
            hg = jnp.maximum(_dot(ug, wug) + _dot(nm, wng) + bG1_r[...], 0.0)
            ug = _dot(hg, wG2_r[...]) + bG2_r[...]
            a0, a1, a2, a3 = e0, e1, e2, e3
        hl = jnp.maximum(_dot(ug, wl1_r[...]) + bl1_r[...], 0.0)
        o_r[...] = _dot(hl, wl2_r[...]) + bl2_r[...]

    return pl.pallas_call(
        body,
        out_shape=jax.ShapeDtypeStruct((B, 15), F32),
    )(u0, nsum, ncnt, wu, wn, bg1, wg2, bg2, scal4,
      wn21, bn21, wn22, bn22,
      we21, be21, we22, be22,
      wE1, bE1, wE2, bE2,
      wX1, bX1, wX2, bX2,
      wG1, bG1, wG2, bG2,
      wl1, bl1, wl2, bl2)


def _row2(v):
    return v.reshape(1, -1)


def kernel(x_p1, ei_p1, ea_p1, y_p1, btc_p1, x_p2, ei_p2, ea_p2, y_p2,
           btc_p2, x_pm, ei_pm, ea_pm, y_pm, btc_pm, Temperature, params):
    p = params
    (we1, be1), (we2, be2) = p['enc_node_1']
    (wee1, bee1), (wee2, bee2) = p['enc_edge_1']
    (wE1, bE1), (wE2, bE2) = p['edge1']
    (wN1, bN1), (wN2, bN2) = p['node1']
    (wG1, bG1), (wG2, bG2) = p['glob1']
    wa, wb, wc = wE1[0:128], wE1[128:256], wE1[256:320]
    wx, wagg = wN1[0:128], wN1[128:192]
    wu, wn = wG1[0:64], wG1[64:192]
    wee1p = jnp.concatenate([wee1, jnp.zeros((4, 64), F32)], axis=0)

    zeros_e = jnp.zeros((NP, 64), F32)
    zeros_n3 = jnp.zeros((G * GACC, 128), F32)
    zc_e = jnp.zeros((NP, 16), F32)
    zc_n3 = jnp.zeros((G * GACC, 16), F32)
    ones16 = jnp.ones((128, 16), F32)

    graphs = ((x_p1, ei_p1, ea_p1, btc_p1),
              (x_p2, ei_p2, ea_p2, btc_p2),
              (x_pm, ei_pm, ea_pm, btc_pm))
    rowo, colo, colu, btco, xps, ea8s = [], [], [], [], [], []
    for g, (x, ei, ea, btc) in enumerate(graphs):
        pad_e = jnp.full((EP - NE,), NN, I32)
        rowo.append(jnp.concatenate([ei[0], pad_e]) + g * NP)
        colo.append(jnp.concatenate([ei[1], pad_e]) + g * NP)
        colu.append(jnp.concatenate([ei[1], pad_e]))
        btco.append(jnp.concatenate(
            [btc, jnp.full((NP - NN,), B, I32)]) + g * GACC)
        xps.append(jnp.pad(x, ((0, NP - NN), (0, 0))))
        ea8s.append(jnp.pad(ea, ((0, EP - NE), (0, 4))))
    rowp = jnp.concatenate(rowo).reshape(G * ECH, 128)
    colp = jnp.concatenate(colo).reshape(G * ECH, 128)
    colpu = jnp.concatenate(colu).reshape(G * ECH, 128)
    btcp = jnp.concatenate(btco).reshape(G * NCH, 128)
    xall = jnp.concatenate(xps)
    ea8all = jnp.concatenate(ea8s)

    ex, a, b = _tc_enc(xall, we1, _row2(be1), we2, _row2(be2), wa, wb)
    u = jnp.full((G * B, 64), 0.1, F32)
    for r in range(2):
        gsum = _sc_gather(a, b, rowp, colp)
        if r == 0:
            ee = _tc_edge0(gsum, ea8all, wee1p, _row2(bee1), wee2,
                           _row2(bee2), wc, _row2(bE1), wE2, _row2(bE2))
            esum, ecnt, ncnt = _sc_scatter_edges(
                ee, colpu, zeros_e, btcp, ones16, zc_e, zc_n3)
        else:
            ee = _tc_edge(gsum, ee, wc, _row2(bE1), wE2, _row2(bE2))
            esum = _sc_scatter_edges(ee, colpu, zeros_e)
        ex, a, b = _tc_node(ex, esum, ecnt, wx, wagg, _row2(bN1), wN2,
                            _row2(bN2), wa, wb)
        nsum = _sc_scatter_nodes(ex, btcp, zeros_n3)
        if r == 0:
            u = _tc_glob(u, nsum, ncnt, wu, wn, _row2(bG1), wG2,
                         _row2(bG2))

    (wn21, bn21), (wn22, bn22) = p['enc_node_2']
    (we21, be21), (we22, be22) = p['enc_edge_2']
    (wE21, bE21), (wE22, bE22) = p['edge2']
    (wX21, bX21), (wX22, bX22) = p['node2']
    (wG21, bG21), (wG22, bG22) = p['glob2']
    (wl1, bl1), (wl2, bl2) = p['last']
    scal4 = jnp.stack([Temperature, y_p1, y_p2, y_pm], axis=1)
    return _tc_phase2(u, nsum, ncnt, wu, wn, _row2(bG1), wG2, _row2(bG2),
                      scal4,
                      wn21, _row2(bn21), wn22, _row2(bn22),
                      we21, _row2(be21), we22, _row2(be22),
                      wE21, _row2(bE21), wE22, _row2(bE22),
                      wX21, _row2(bX21), wX22, _row2(bX22),
                      wG21, _row2(bG21), wG22, _row2(bG22),
                      wl1, _row2(bl1), wl2, _row2(bl2))
